# baseline (speedup 1.0000x reference)
import functools

import jax
import jax.numpy as jnp
from jax import lax
from jax.experimental import pallas as pl
from jax.experimental.pallas import tpu as pltpu
from jax.experimental.pallas import tpu_sc as plsc

DIM = 64
ROWS = 16384
COLS = 26
B_TOTAL = ROWS * COLS
NUM_WORKERS = 32
B_PER_W = B_TOTAL // NUM_WORKERS
CHUNK = 128
N_CHUNKS = B_PER_W // CHUNK

_mesh = plsc.VectorSubcoreMesh(core_axis_name="c", subcore_axis_name="s")


@functools.partial(
    pl.kernel,
    mesh=_mesh,
    out_type=jax.ShapeDtypeStruct((B_TOTAL, DIM), jnp.float32),
    scratch_types=[
        pltpu.VMEM((N_CHUNKS, CHUNK), jnp.int32),
        pltpu.VMEM((CHUNK, DIM), jnp.float32),
        pltpu.SemaphoreType.DMA,
    ],
    compiler_params=pltpu.CompilerParams(use_tc_tiling_on_sc=False),
)
def _gather_kernel(idx_hbm, table_hbm, out_hbm, idx_v, rows_v, sem):
    wid = lax.axis_index("s") * 2 + lax.axis_index("c")
    base = wid * B_PER_W
    pltpu.sync_copy(idx_hbm.at[wid], idx_v)

    def body(i, carry):
        pltpu.async_copy(table_hbm.at[idx_v.at[i]], rows_v, sem).wait()
        pltpu.sync_copy(rows_v, out_hbm.at[pl.ds(base + i * CHUNK, CHUNK)])
        return carry

    lax.fori_loop(0, N_CHUNKS, body, 0)


def kernel(x, table):
    idx = x.reshape(NUM_WORKERS, N_CHUNKS, CHUNK)
    out = _gather_kernel(idx, table)
    return out.reshape(ROWS, COLS, DIM)


# speedup vs baseline: 1.0756x; 1.0756x over previous
import functools

import jax
import jax.numpy as jnp
from jax import lax
from jax.experimental import pallas as pl
from jax.experimental.pallas import tpu as pltpu
from jax.experimental.pallas import tpu_sc as plsc

DIM = 64
ROWS = 16384
COLS = 26
B_TOTAL = ROWS * COLS
NUM_WORKERS = 32
B_PER_W = B_TOTAL // NUM_WORKERS
CHUNK = 128
N_CHUNKS = B_PER_W // CHUNK
K = 2
GROUP = K * CHUNK
N_GROUPS = N_CHUNKS // K
NBUF = 4

_mesh = plsc.VectorSubcoreMesh(core_axis_name="c", subcore_axis_name="s")


@functools.partial(
    pl.kernel,
    mesh=_mesh,
    out_type=jax.ShapeDtypeStruct((B_TOTAL, DIM), jnp.float32),
    scratch_types=[
        pltpu.VMEM((N_CHUNKS, CHUNK), jnp.int32),
        pltpu.VMEM((NBUF, GROUP, DIM), jnp.float32),
    ]
    + [pltpu.SemaphoreType.DMA] * NBUF
    + [pltpu.SemaphoreType.DMA] * NBUF,
    compiler_params=pltpu.CompilerParams(use_tc_tiling_on_sc=False),
)
def _gather_kernel(idx_hbm, table_hbm, out_hbm, idx_v, rows_v, *sems):
    gsems = sems[:NBUF]
    ssems = sems[NBUF:]
    wid = lax.axis_index("s") * 2 + lax.axis_index("c")
    base = wid * B_PER_W
    pltpu.sync_copy(idx_hbm.at[wid], idx_v)

    def fire_gathers(g, b):
        for j in range(K):
            pltpu.async_copy(
                table_hbm.at[idx_v.at[g * K + j]],
                rows_v.at[b, pl.ds(j * CHUNK, CHUNK)],
                gsems[b],
            )

    def drain_gathers(b):
        pltpu.make_async_copy(
            table_hbm.at[pl.ds(0, GROUP)], rows_v.at[b], gsems[b]
        ).wait()

    def fire_store(g, b):
        pltpu.async_copy(
            rows_v.at[b], out_hbm.at[pl.ds(base + g * GROUP, GROUP)], ssems[b]
        )

    def drain_store(b):
        pltpu.make_async_copy(
            rows_v.at[b], out_hbm.at[pl.ds(base, GROUP)], ssems[b]
        ).wait()

    for b in range(NBUF):
        fire_gathers(b, b)

    def outer(o, carry):
        g0 = o * NBUF
        for b in range(NBUF):
            g = g0 + b
            drain_gathers(b)
            fire_store(g, b)
            drain_store(b)
            fire_gathers(g + NBUF, b)
        return carry

    lax.fori_loop(0, (N_GROUPS - NBUF) // NBUF, outer, 0)

    for b in range(NBUF):
        g = N_GROUPS - NBUF + b
        drain_gathers(b)
        fire_store(g, b)
    for b in range(NBUF):
        drain_store(b)


def kernel(x, table):
    idx = x.reshape(NUM_WORKERS, N_CHUNKS, CHUNK)
    out = _gather_kernel(idx, table)
    return out.reshape(ROWS, COLS, DIM)


# speedup vs baseline: 1.1218x; 1.0429x over previous
import functools

import jax
import jax.numpy as jnp
from jax import lax
from jax.experimental import pallas as pl
from jax.experimental.pallas import tpu as pltpu
from jax.experimental.pallas import tpu_sc as plsc

DIM = 64
ROWS = 16384
COLS = 26
B_TOTAL = ROWS * COLS
NUM_WORKERS = 32
B_PER_W = B_TOTAL // NUM_WORKERS
CHUNK = 128
N_CHUNKS = B_PER_W // CHUNK
K = 2
GROUP = K * CHUNK
N_GROUPS = N_CHUNKS // K
NBUF = 4

_mesh = plsc.VectorSubcoreMesh(core_axis_name="c", subcore_axis_name="s")


@functools.partial(
    pl.kernel,
    mesh=_mesh,
    out_type=jax.ShapeDtypeStruct((B_TOTAL, DIM), jnp.float32),
    scratch_types=[
        pltpu.VMEM((N_CHUNKS, CHUNK), jnp.int32),
        pltpu.VMEM((NBUF, GROUP, DIM), jnp.float32),
    ]
    + [pltpu.SemaphoreType.DMA] * NBUF
    + [pltpu.SemaphoreType.DMA] * NBUF,
    compiler_params=pltpu.CompilerParams(use_tc_tiling_on_sc=False),
)
def _gather_kernel(idx_hbm, table_hbm, out_hbm, idx_v, rows_v, *sems):
    gsems = sems[:NBUF]
    ssems = sems[NBUF:]
    wid = lax.axis_index("s") * 2 + lax.axis_index("c")
    base = wid * B_PER_W
    pltpu.sync_copy(idx_hbm.at[wid], idx_v)

    def fire_gathers(g, b):
        for j in range(K):
            pltpu.async_copy(
                table_hbm.at[idx_v.at[g * K + j]],
                rows_v.at[b, pl.ds(j * CHUNK, CHUNK)],
                gsems[b],
            )

    def drain_gathers(b):
        pltpu.make_async_copy(
            table_hbm.at[pl.ds(0, GROUP)], rows_v.at[b], gsems[b]
        ).wait()

    def fire_store(g, b):
        pltpu.async_copy(
            rows_v.at[b], out_hbm.at[pl.ds(base + g * GROUP, GROUP)], ssems[b]
        )

    def drain_store(b):
        pltpu.make_async_copy(
            rows_v.at[b], out_hbm.at[pl.ds(base, GROUP)], ssems[b]
        ).wait()

    for b in range(NBUF):
        fire_gathers(b, b)

    def outer(o, carry):
        g0 = o * NBUF
        for b in range(NBUF):
            g = g0 + b
            drain_gathers(b)
            fire_store(g, b)
            drain_store(b)
            fire_gathers(g + NBUF, b)
        return carry

    lax.fori_loop(0, (N_GROUPS - NBUF) // NBUF, outer, 0)

    for b in range(NBUF):
        g = N_GROUPS - NBUF + b
        drain_gathers(b)
        fire_store(g, b)
    for b in range(NBUF):
        drain_store(b)


def kernel(x, table):
    idx = x.T.reshape(NUM_WORKERS, N_CHUNKS, CHUNK)
    out = _gather_kernel(idx, table)
    return out.reshape(COLS, ROWS, DIM).transpose(1, 0, 2)


# speedup vs baseline: 1.3373x; 1.1921x over previous
import functools

import jax
import jax.numpy as jnp
from jax import lax
from jax.experimental import pallas as pl
from jax.experimental.pallas import tpu as pltpu
from jax.experimental.pallas import tpu_sc as plsc

DIM = 64
DIMP = 128
ROWS = 16384
COLS = 26
B_TOTAL = ROWS * COLS
NUM_WORKERS = 32
B_PER_W = B_TOTAL // NUM_WORKERS
CHUNK = 128
N_CHUNKS = B_PER_W // CHUNK
GROUPS_PER_ROW = ROWS // CHUNK
NBUF = 4
NST = 2

_mesh = plsc.VectorSubcoreMesh(core_axis_name="c", subcore_axis_name="s")


@functools.partial(
    pl.kernel,
    mesh=_mesh,
    out_type=jax.ShapeDtypeStruct((COLS, ROWS, DIM), jnp.float32),
    scratch_types=[
        pltpu.VMEM((N_CHUNKS, CHUNK), jnp.int32),
        pltpu.VMEM((NBUF, CHUNK, DIMP), jnp.float32),
        pltpu.VMEM((NST, CHUNK, DIM), jnp.float32),
    ]
    + [pltpu.SemaphoreType.DMA] * NBUF
    + [pltpu.SemaphoreType.DMA] * NST,
    compiler_params=pltpu.CompilerParams(use_tc_tiling_on_sc=True),
)
def _gather_kernel(idx_hbm, table_hbm, out_hbm, idx_v, rows_v, st_v, *sems):
    gsems = sems[:NBUF]
    ssems = sems[NBUF:]
    wid = lax.axis_index("s") * 2 + lax.axis_index("c")
    g_base = wid * N_CHUNKS
    pltpu.sync_copy(idx_hbm.at[wid], idx_v)

    def fire_gather(g, b):
        pltpu.async_copy(table_hbm.at[idx_v.at[g]], rows_v.at[b], gsems[b])

    def drain_gather(b):
        pltpu.make_async_copy(
            table_hbm.at[pl.ds(0, CHUNK)], rows_v.at[b], gsems[b]
        ).wait()

    def compact(b, s):
        def row_block(r0, carry):
            for rr in range(8):
                r = r0 * 8 + rr
                for k in range(DIM // 16):
                    st_v[s, r, pl.ds(k * 16, 16)] = rows_v[b, r, pl.ds(k * 16, 16)]
            return carry

        lax.fori_loop(0, CHUNK // 8, row_block, 0)

    def fire_store(g, s):
        gg = g_base + g
        c = gg // GROUPS_PER_ROW
        b0 = (gg % GROUPS_PER_ROW) * CHUNK
        pltpu.async_copy(
            st_v.at[s],
            out_hbm.at[c, pl.ds(b0, CHUNK)],
            ssems[s],
        )

    def drain_store(s):
        pltpu.make_async_copy(
            st_v.at[s],
            out_hbm.at[0, pl.ds(0, CHUNK)],
            ssems[s],
        ).wait()

    for b in range(NBUF):
        fire_gather(b, b)

    for b in range(NBUF):
        s = b % NST
        drain_gather(b)
        if b >= NST:
            drain_store(s)
        compact(b, s)
        fire_store(b, s)
        fire_gather(b + NBUF, b)

    def outer(o, carry):
        g0 = o * NBUF
        for b in range(NBUF):
            g = g0 + b
            s = b % NST
            drain_gather(b)
            drain_store(s)
            compact(b, s)
            fire_store(g, s)
            fire_gather(g + NBUF, b)
        return carry

    lax.fori_loop(1, N_CHUNKS // NBUF - 1, outer, 0)

    for b in range(NBUF):
        g = N_CHUNKS - NBUF + b
        s = b % NST
        drain_gather(b)
        drain_store(s)
        compact(b, s)
        fire_store(g, s)
    for s in range(NST):
        drain_store(s)


def kernel(x, table):
    idx = x.T.reshape(NUM_WORKERS, N_CHUNKS, CHUNK)
    tpad = jnp.pad(table, ((0, 0), (0, DIMP - DIM)))
    out = _gather_kernel(idx, tpad)
    return out.transpose(1, 0, 2)
